# Initial kernel scaffold; baseline (speedup 1.0000x reference)
import jax
import jax.numpy as jnp
from jax.experimental import pallas as pl


def kernel(x, gemm_edges, feature_values, params):
    raise NotImplementedError("write your pallas kernel here")


# speedup vs baseline: 3.2018x; 3.2018x over previous
import functools
import math

import jax
import jax.numpy as jnp
from jax import lax
from jax.experimental import pallas as pl
from jax.experimental.pallas import tpu as pltpu
from jax.experimental.pallas import tpu_sc as plsc

K = [5, 64, 128, 128, 128]
RES = [50000, 35000, 20000, 10000, 5000]
NB = 2
EPS = 1e-5
GROUPS = 16
NWORKERS = 32


def _pcall(*a, **kw):
    return pl.pallas_call(*a, **kw)


def _gather_rows(table, idx):
    T, C = table.shape
    (M,) = idx.shape
    chunk = max(8, min(512, (96 * 1024 // (C * 4)) // 8 * 8))
    per_w = max(chunk, -(-M // NWORKERS))
    per_w = -(-per_w // chunk) * chunk
    m_pad = per_w * NWORKERS
    steps = per_w // chunk
    if m_pad != M:
        idx = jnp.concatenate([idx, jnp.zeros((m_pad - M,), jnp.int32)])

    mesh = plsc.VectorSubcoreMesh(core_axis_name="c", subcore_axis_name="s")

    @functools.partial(
        pl.kernel,
        mesh=mesh,
        out_type=jax.ShapeDtypeStruct((m_pad, C), jnp.float32),
        scratch_types=[
            pltpu.VMEM((chunk,), jnp.int32),
            pltpu.VMEM((chunk, C), jnp.float32),
            pltpu.SemaphoreType.DMA,
        ],
    )
    def gk(table_hbm, idx_hbm, out_hbm, idx_v, rows_v, sem):
        wid = lax.axis_index("s") * 2 + lax.axis_index("c")
        base = wid * per_w

        def step(j, carry):
            off = base + j * chunk
            pltpu.sync_copy(idx_hbm.at[pl.ds(off, chunk)], idx_v)
            pltpu.async_copy(table_hbm.at[idx_v], rows_v, sem).wait()
            pltpu.sync_copy(rows_v, out_hbm.at[pl.ds(off, chunk)])
            return carry

        lax.fori_loop(0, steps, step, 0)

    out = gk(table, idx)
    return out[:M] if m_pad != M else out


def _conv(x, g, wcat, bn_scale=None, bn_shift=None, res=None, block=1000):
    N, cin = x.shape
    cout = wcat.shape[1]
    R = block
    assert N % R == 0
    bn = bn_scale is not None
    has_res = res is not None

    def body(*refs):
        x_ref, g_ref, w_ref = refs[0], refs[1], refs[2]
        rest = list(refs[3:-1])
        out_ref = refs[-1]
        if bn:
            sc = rest.pop(0)[0, :]
            sh = rest.pop(0)[0, :]
            f = lambda v: jnp.maximum(v, 0.0) * sc[None, :] + sh[None, :]
        else:
            f = lambda v: v
        xb = f(x_ref[...])
        a = f(g_ref[0])
        b = f(g_ref[1])
        c = f(g_ref[2])
        d = f(g_ref[3])
        taps = jnp.concatenate(
            [xb, a + c, b + d, jnp.abs(a - c), jnp.abs(b - d)], axis=1
        )
        acc = jnp.dot(taps, w_ref[...], preferred_element_type=jnp.float32)
        if has_res:
            acc = jnp.maximum(acc + rest.pop(0)[...], 0.0)
        out_ref[...] = acc

    in_specs = [
        pl.BlockSpec((R, cin), lambda i: (i, 0)),
        pl.BlockSpec((4, R, cin), lambda i: (0, i, 0)),
        pl.BlockSpec((5 * cin, cout), lambda i: (0, 0)),
    ]
    args = [x, g, wcat]
    if bn:
        in_specs += [
            pl.BlockSpec((1, cin), lambda i: (0, 0)),
            pl.BlockSpec((1, cin), lambda i: (0, 0)),
        ]
        args += [bn_scale[None, :], bn_shift[None, :]]
    if has_res:
        in_specs.append(pl.BlockSpec((R, cout), lambda i: (i, 0)))
        args.append(res)
    return _pcall(
        body,
        grid=(N // R,),
        in_specs=in_specs,
        out_specs=pl.BlockSpec((R, cout), lambda i: (i, 0)),
        out_shape=jax.ShapeDtypeStruct((N, cout), jnp.float32),
    )(*args)


def _bn_stats(y, block=1000):
    N, C = y.shape
    R = block

    def body(y_ref, out_ref):
        i = pl.program_id(0)
        t = jnp.maximum(y_ref[...], 0.0)
        blk = jnp.stack([jnp.sum(t, axis=0), jnp.sum(t * t, axis=0)], axis=0)

        @pl.when(i == 0)
        def _():
            out_ref[...] = blk

        @pl.when(i > 0)
        def _():
            out_ref[...] += blk

    return _pcall(
        body,
        grid=(N // R,),
        in_specs=[pl.BlockSpec((R, C), lambda i: (i, 0))],
        out_specs=pl.BlockSpec((2, C), lambda i: (0, 0)),
        out_shape=jax.ShapeDtypeStruct((2, C), jnp.float32),
    )(y)


def _gn_stats(x, E, block=1000):
    N, C = x.shape
    R = block
    J = E // R

    def body(x_ref, out_ref):
        j = pl.program_id(1)
        xb = x_ref[...]
        blk = jnp.stack([jnp.sum(xb, axis=0), jnp.sum(xb * xb, axis=0)], axis=0)

        @pl.when(j == 0)
        def _():
            out_ref[...] = blk[None]

        @pl.when(j > 0)
        def _():
            out_ref[...] += blk[None]

    return _pcall(
        body,
        grid=(NB, J),
        in_specs=[pl.BlockSpec((R, C), lambda b, j: (b * J + j, 0))],
        out_specs=pl.BlockSpec((1, 2, C), lambda b, j: (b, 0, 0)),
        out_shape=jax.ShapeDtypeStruct((NB, 2, C), jnp.float32),
    )(x)


def _gn_apply(x, gscale, gshift, E, block=1000):
    N, C = x.shape
    R = block
    J = E // R

    def body(x_ref, sc_ref, sh_ref, z_ref, n_ref):
        z = jnp.maximum(x_ref[...] * sc_ref[0, 0][None, :] + sh_ref[0, 0][None, :], 0.0)
        z_ref[...] = z
        n_ref[...] = jnp.sum(z * z, axis=1, keepdims=True)

    z, norms = _pcall(
        body,
        grid=(NB, J),
        in_specs=[
            pl.BlockSpec((R, C), lambda b, j: (b * J + j, 0)),
            pl.BlockSpec((1, 1, C), lambda b, j: (b, 0, 0)),
            pl.BlockSpec((1, 1, C), lambda b, j: (b, 0, 0)),
        ],
        out_specs=[
            pl.BlockSpec((R, C), lambda b, j: (b * J + j, 0)),
            pl.BlockSpec((R, 1), lambda b, j: (b * J + j, 0)),
        ],
        out_shape=[
            jax.ShapeDtypeStruct((N, C), jnp.float32),
            jax.ShapeDtypeStruct((N, 1), jnp.float32),
        ],
    )(x, gscale[:, None, :], gshift[:, None, :])
    return z, norms.reshape(NB, E)


def _mean_fc(x, E, w1, b1, w2, b2):
    N, C = x.shape

    def mbody(x_ref, out_ref):
        out_ref[...] = jnp.mean(x_ref[...], axis=0)[None, None]

    xm = _pcall(
        mbody,
        grid=(NB,),
        in_specs=[pl.BlockSpec((E, C), lambda b: (b, 0))],
        out_specs=pl.BlockSpec((1, 1, C), lambda b: (b, 0, 0)),
        out_shape=jax.ShapeDtypeStruct((NB, 1, C), jnp.float32),
    )(x).reshape(NB, C)

    f, ncls = w1.shape[1], w2.shape[1]

    def fcbody(x_ref, w1_ref, b1_ref, w2_ref, b2_ref, out_ref):
        h = jnp.maximum(
            jnp.dot(x_ref[...], w1_ref[...], preferred_element_type=jnp.float32)
            + b1_ref[...],
            0.0,
        )
        out_ref[...] = (
            jnp.dot(h, w2_ref[...], preferred_element_type=jnp.float32) + b2_ref[...]
        )

    return _pcall(
        fcbody,
        out_shape=jax.ShapeDtypeStruct((NB, ncls), jnp.float32),
    )(xm, w1, b1[None, :], w2, b2[None, :])


def _neighbor_idx(gemm, E):
    boff = (jnp.arange(NB, dtype=jnp.int32) * E)[None, :, None]
    idx = gemm.astype(jnp.int32).transpose(2, 0, 1) + boff
    return idx.reshape(-1)


CP = 128


def _wcat_p(w):
    cout, cin, _ = w.shape
    wt = jnp.transpose(w, (2, 1, 0))
    wt = jnp.pad(wt, ((0, 0), (0, CP - cin), (0, CP - cout)))
    return wt.reshape(5 * CP, CP)


def _padc(v):
    return jnp.pad(v, (0, CP - v.shape[0]))


def kernel(x, gemm_edges, feature_values, params):
    del feature_values
    B, C0, E = x.shape
    feat = jnp.transpose(x, (0, 2, 1)).reshape(B * E, C0)
    feat = jnp.concatenate([feat, jnp.zeros((B * E, CP - C0), jnp.float32)], axis=1)
    gemm = gemm_edges.astype(jnp.int32)

    for i in range(4):
        blk = params["block%d" % i]
        E = RES[i]
        N = B * E
        cout = K[i + 1]
        w0 = _wcat_p(blk["w0"])
        w1 = _wcat_p(blk["w1"])

        nidx = _neighbor_idx(gemm, E)
        g0 = _gather_rows(feat, nidx).reshape(4, N, CP)
        y = _conv(feat, g0, w0)

        st = _bn_stats(y)
        mean = st[0] / N
        var = st[1] / N - mean * mean
        bscale = _padc(blk["bn_g1"]) * lax.rsqrt(var + EPS)
        bshift = _padc(blk["bn_b1"]) - mean * bscale

        g1 = _gather_rows(y, nidx).reshape(4, N, CP)
        x2 = _conv(y, g1, w1, bn_scale=bscale, bn_shift=bshift, res=y)

        gst = _gn_stats(x2, E)
        cg = cout // GROUPS
        gs = gst[:, :, :cout].reshape(NB, 2, GROUPS, cg).sum(axis=3)
        gm = gs[:, 0] / (cg * E)
        gv = gs[:, 1] / (cg * E) - gm * gm
        grs = lax.rsqrt(gv + EPS)
        gscale = blk["gn_g"][None, :] * jnp.repeat(grs, cg, axis=1)
        gshift = blk["gn_b"][None, :] - jnp.repeat(gm, cg, axis=1) * gscale
        gscale = jnp.pad(gscale, ((0, 0), (0, CP - cout)))
        gshift = jnp.pad(gshift, ((0, 0), (0, CP - cout)))

        z, norms = _gn_apply(x2, gscale, gshift, E)

        target = RES[i + 1]
        _, keep = lax.top_k(norms, target)
        keep = jnp.sort(keep, axis=1).astype(jnp.int32)
        kflat = (keep + (jnp.arange(NB, dtype=jnp.int32) * E)[:, None]).reshape(-1)
        feat = _gather_rows(z, kflat)

        def remap(gb, kb):
            m = jnp.full((E,), -1, jnp.int32).at[kb].set(
                jnp.arange(target, dtype=jnp.int32)
            )
            ng = m[gb[kb]]
            sn = jnp.broadcast_to(
                jnp.arange(target, dtype=jnp.int32)[:, None], ng.shape
            )
            return jnp.where(ng < 0, sn, ng)

        gemm = jax.vmap(remap)(gemm, keep)

    return _mean_fc(
        feat,
        RES[4],
        params["fc1_w"].T,
        params["fc1_b"],
        params["fc2_w"].T,
        params["fc2_b"],
    )
